# baseline (speedup 1.0000x reference)
import functools

import jax
import jax.numpy as jnp
from jax import lax
from jax.experimental import pallas as pl
from jax.experimental.pallas import tpu as pltpu
from jax.experimental.pallas import tpu_sc as plsc

USER_NUM = 20000
N_NODES = 50000
EMBED = 64
NUM_LAYERS = 3
BATCH = 4096

NP = 50176
HALF = NP // 2
TRASH = 16
ACC_ROWS = HALF + TRASH
Z_SLICE = ACC_ROWS // 16
E_PAD = 802816
CHUNK = 128
EDGE_ROWS = E_PAD // 128
PROWS = EDGE_ROWS // 32
REGION_ROWS = PROWS + 4
NREGIONS = 64


_LANE_DNUMS = lax.GatherDimensionNumbers(
    offset_dims=(), collapsed_slice_dims=(0,), start_index_map=(0,))


def _lane_bcast(vec, lane):
    idx = jnp.full((16, 1), lane, jnp.int32)
    return lax.gather(vec, idx, _LANE_DNUMS, (1,),
                      mode=lax.GatherScatterMode.PROMISE_IN_BOUNDS)


def _part_body(edata_hbm, edp_hbm, cnt_hbm, ebuf, stg, cbuf, sem_i):
    c = lax.axis_index("c")
    s = lax.axis_index("s")
    w = s * 2 + c
    base = w * PROWS
    half_i = jnp.full((16,), HALF, jnp.int32)
    zero_v = jnp.zeros((16,), jnp.int32)

    def chunk_body(t, carry):
        pos0, pos1, row0, row1 = carry
        pltpu.async_copy(edata_hbm.at[base + t], ebuf, sem_i).wait()
        poss = [pos0, pos1]
        rows = [row0, row1]
        for h in range(2):
            pos = poss[h]
            for k in range(8):
                sv = ebuf[0, pl.ds(16 * k, 16)]
                dv = ebuf[1, pl.ds(16 * k, 16)] - (h * HALF)
                wv = ebuf[2, pl.ds(16 * k, 16)]
                ok = (dv >= 0) & (dv < half_i)
                plsc.store_compressed(stg.at[h, 0, pl.ds(pos, 16)], sv, mask=ok)
                plsc.store_compressed(stg.at[h, 1, pl.ds(pos, 16)], dv, mask=ok)
                plsc.store_compressed(stg.at[h, 2, pl.ds(pos, 16)], wv, mask=ok)
                pos = pos + jnp.max(plsc.all_reduce_population_count(ok))
            flush = pos >= 128
            rdst = (h * 32 + w) * REGION_ROWS + rows[h]

            @pl.when(flush)
            def _():
                pltpu.sync_copy(stg.at[h, :, pl.ds(0, 128)], edp_hbm.at[rdst])
                for a in range(3):
                    for i in range(16):
                        stg[h, a, pl.ds(16 * i, 16)] = stg[h, a, pl.ds(128 + 16 * i, 16)]

            poss[h] = jnp.where(flush, pos - 128, pos)
            rows[h] = jnp.where(flush, rows[h] + 1, rows[h])
        return (poss[0], poss[1], rows[0], rows[1])

    z32 = jnp.zeros((), jnp.int32)
    pos0, pos1, row0, row1 = lax.fori_loop(
        0, PROWS, chunk_body, (z32, z32, z32, z32))

    poss = [pos0, pos1]
    rowss = [row0, row1]
    for h in range(2):
        pos, row = poss[h], rowss[h]
        for a in range(3):
            for i in range(8):
                stg[h, a, pl.ds(pos + 16 * i, 16)] = zero_v
        rbase = (h * 32 + w) * REGION_ROWS
        pltpu.sync_copy(stg.at[h, :, pl.ds(0, 128)], edp_hbm.at[rbase + row])
        row = row + 1
        for a in range(3):
            for i in range(8):
                stg[h, a, pl.ds(16 * i, 16)] = zero_v
        for extra in range(3):
            pltpu.sync_copy(stg.at[h, :, pl.ds(0, 128)],
                            edp_hbm.at[rbase + row + extra])
        count = row + (row & 1)
        for j in range(1):
            cbuf[pl.ds(0, 16)] = jnp.full((16,), 0, jnp.int32) + count
        pltpu.sync_copy(cbuf, cnt_hbm.at[h * 32 + w])


@functools.lru_cache(maxsize=None)
def _make_part():
    mesh = plsc.VectorSubcoreMesh(core_axis_name="c", subcore_axis_name="s")
    return pl.kernel(
        _part_body,
        out_type=(jax.ShapeDtypeStruct((NREGIONS * REGION_ROWS, 3, 128), jnp.int32),
                  jax.ShapeDtypeStruct((NREGIONS, 16), jnp.int32)),
        mesh=mesh,
        scratch_types=[
            pltpu.VMEM((3, 128), jnp.int32),
            pltpu.VMEM((2, 3, 384), jnp.int32),
            pltpu.VMEM((16,), jnp.int32),
            pltpu.SemaphoreType.DMA,
        ],
        compiler_params=pltpu.CompilerParams(use_tc_tiling_on_sc=False, needs_layout_passes=False),
        name="lgcn_partition",
    )


def _layer_body(scale, x_hbm, edp_hbm, cnt_hbm, out_hbm,
                ev0, ev1, rows0, rows1, cbuf, acc, sem_i, sem_g0, sem_g1):
    c = lax.axis_index("c")
    s = lax.axis_index("s")

    def zz(e, _):
        z = jnp.zeros((16,), jnp.float32)
        for j in range(EMBED // 16):
            rows0[e, pl.ds(16 * j, 16)] = z
        return 0
    lax.fori_loop(0, CHUNK, zz, 0)
    for i in range(Z_SLICE // CHUNK):
        pltpu.sync_copy(rows0, acc.at[pl.ds(s * Z_SLICE + i * CHUNK, CHUNK)])
    rem = Z_SLICE % CHUNK
    if rem:
        pltpu.sync_copy(rows0.at[pl.ds(0, rem)],
                        acc.at[pl.ds(s * Z_SLICE + (Z_SLICE // CHUNK) * CHUNK, rem)])
    plsc.subcore_barrier()

    bufs = ((ev0, rows0, sem_g0), (ev1, rows1, sem_g1))

    for ri in range(2):
        reg = s * 2 + ri
        rbase = (c * 32 + reg) * REGION_ROWS
        pltpu.sync_copy(cnt_hbm.at[c * 32 + reg], cbuf)
        nchunks = jnp.max(cbuf[pl.ds(0, 16)])

        pltpu.async_copy(edp_hbm.at[rbase], ev0, sem_i).wait()
        pltpu.async_copy(x_hbm.at[ev0.at[0].at[pl.ds(0, 64)]], rows0.at[pl.ds(0, 64)], sem_g0)
        pltpu.async_copy(x_hbm.at[ev0.at[0].at[pl.ds(64, 64)]], rows0.at[pl.ds(64, 64)], sem_g0)
        pltpu.async_copy(edp_hbm.at[rbase + 1], ev1, sem_i)

        def step(t, cur, nxt):
            ebuf, rows, sem_g = cur
            ebuf_n, rows_n, sem_g_n = nxt
            pltpu.make_async_copy(edp_hbm.at[rbase + t + 1], ebuf_n, sem_i).wait()
            pltpu.async_copy(x_hbm.at[ebuf_n.at[0].at[pl.ds(0, 64)]], rows_n.at[pl.ds(0, 64)], sem_g_n)
            pltpu.async_copy(x_hbm.at[ebuf_n.at[0].at[pl.ds(64, 64)]], rows_n.at[pl.ds(64, 64)], sem_g_n)
            wvecs = [plsc.bitcast(ebuf[2, pl.ds(16 * b, 16)], jnp.float32) * scale
                     for b in range(8)]
            pltpu.make_async_copy(x_hbm.at[pl.ds(0, 64)], rows.at[pl.ds(0, 64)], sem_g).wait()
            pltpu.make_async_copy(x_hbm.at[pl.ds(0, 64)], rows.at[pl.ds(64, 64)], sem_g).wait()
            for b in range(8):
                for l in range(16):
                    wb = _lane_bcast(wvecs[b], l)
                    e = 16 * b + l
                    for j in range(EMBED // 16):
                        rows[e, pl.ds(16 * j, 16)] = rows[e, pl.ds(16 * j, 16)] * wb
            pltpu.sync_copy(rows, acc.at[ebuf.at[1]], add=True)
            pltpu.async_copy(edp_hbm.at[rbase + t + 2], ebuf, sem_i)

        def pair_body(i, _):
            t = i * 2
            step(t, bufs[0], bufs[1])
            step(t + 1, bufs[1], bufs[0])
            return 0
        lax.fori_loop(0, 1, pair_body, 0)
        lax.fori_loop(1, lax.max(nchunks, 2) // 2, pair_body, 0)

        pltpu.make_async_copy(x_hbm.at[pl.ds(0, 64)], rows0.at[pl.ds(0, 64)], sem_g0).wait()
        pltpu.make_async_copy(x_hbm.at[pl.ds(0, 64)], rows0.at[pl.ds(64, 64)], sem_g0).wait()
        pltpu.make_async_copy(edp_hbm.at[rbase], ev1, sem_i).wait()

    plsc.subcore_barrier()

    wb_rows = HALF // 16
    pltpu.sync_copy(acc.at[pl.ds(s * wb_rows, wb_rows)],
                    out_hbm.at[pl.ds(c * HALF + s * wb_rows, wb_rows)])


@functools.lru_cache(maxsize=None)
def _make_layer(scale):
    mesh = plsc.VectorSubcoreMesh(core_axis_name="c", subcore_axis_name="s")
    return pl.kernel(
        functools.partial(_layer_body, scale),
        out_type=jax.ShapeDtypeStruct((NP, EMBED), jnp.float32),
        mesh=mesh,
        scratch_types=[
            pltpu.VMEM((3, 128), jnp.int32),
            pltpu.VMEM((3, 128), jnp.int32),
            pltpu.VMEM((CHUNK, EMBED), jnp.float32),
            pltpu.VMEM((CHUNK, EMBED), jnp.float32),
            pltpu.VMEM((16,), jnp.int32),
            pltpu.VMEM_SHARED((ACC_ROWS, EMBED), jnp.float32),
            pltpu.SemaphoreType.DMA,
            pltpu.SemaphoreType.DMA,
            pltpu.SemaphoreType.DMA,
        ],
        compiler_params=pltpu.CompilerParams(use_tc_tiling_on_sc=False, needs_layout_passes=False),
        name=f"lgcn_layer_{int(1.0/scale)}",
    )


def _final_body(x0, x1, x2, x3, uidx_hbm, iidx_hbm, u_hbm, i_hbm,
                idxv, g0, g1, g2, g3, sem):
    c = lax.axis_index("c")
    s = lax.axis_index("s")
    wid = s * 2 + c
    base = wid * (BATCH // 32)

    def do(idx_hbm, off, out_hbm):
        pltpu.sync_copy(idx_hbm.at[pl.ds(base, BATCH // 32)], idxv)
        if off:
            offv = jnp.full((16,), off, jnp.int32)
            for k in range(BATCH // 32 // 16):
                idxv[pl.ds(16 * k, 16)] = idxv[pl.ds(16 * k, 16)] + offv
        cps = [pltpu.async_copy(x.at[idxv], g, sem)
               for x, g in ((x0, g0), (x1, g1), (x2, g2), (x3, g3))]
        for cp in cps:
            cp.wait()

        def sum_body(e, _):
            for j in range(EMBED // 16):
                d = pl.ds(16 * j, 16)
                g0[e, d] = g0[e, d] + g1[e, d] + g2[e, d] + g3[e, d]
            return 0
        lax.fori_loop(0, BATCH // 32, sum_body, 0)
        pltpu.sync_copy(g0, out_hbm.at[pl.ds(base, BATCH // 32)])

    do(uidx_hbm, 0, u_hbm)
    do(iidx_hbm, USER_NUM, i_hbm)


@functools.lru_cache(maxsize=None)
def _make_final():
    mesh = plsc.VectorSubcoreMesh(core_axis_name="c", subcore_axis_name="s")
    return pl.kernel(
        _final_body,
        out_type=(jax.ShapeDtypeStruct((BATCH, EMBED), jnp.float32),
                  jax.ShapeDtypeStruct((BATCH, EMBED), jnp.float32)),
        mesh=mesh,
        scratch_types=[
            pltpu.VMEM((BATCH // 32,), jnp.int32),
            pltpu.VMEM((BATCH // 32, EMBED), jnp.float32),
            pltpu.VMEM((BATCH // 32, EMBED), jnp.float32),
            pltpu.VMEM((BATCH // 32, EMBED), jnp.float32),
            pltpu.VMEM((BATCH // 32, EMBED), jnp.float32),
            pltpu.SemaphoreType.DMA,
        ],
        compiler_params=pltpu.CompilerParams(use_tc_tiling_on_sc=False, needs_layout_passes=False),
        name="lgcn_final_gather",
    )


def _mlp_body(u_ref, i_ref, w1u_ref, w1i_ref, b1_ref, w2_ref, b2_ref, w3_ref, b3_ref, o_ref):
    h = jnp.dot(u_ref[...], w1u_ref[...], preferred_element_type=jnp.float32)
    h += jnp.dot(i_ref[...], w1i_ref[...], preferred_element_type=jnp.float32)
    h = jax.nn.relu(h + b1_ref[...])
    h2 = jnp.dot(h, w2_ref[...], preferred_element_type=jnp.float32) + b2_ref[...]
    o_ref[...] = jnp.dot(h2, w3_ref[...], preferred_element_type=jnp.float32) + b3_ref[...]


def _mlp(u, i, W1, b1, W2, b2, W3, b3):
    out = pl.pallas_call(
        _mlp_body,
        out_shape=jax.ShapeDtypeStruct((BATCH, 1), jnp.float32),
    )(u, i, W1[:EMBED], W1[EMBED:], b1[None, :], W2, b2[None, :], W3, b3[None, :])
    return out.reshape(-1)


def kernel(userIdx, itemIdx, edge_index, edge_weight, emb_user, emb_item, W1, b1, W2, b2, W3, b3):
    n_edges = edge_weight.shape[0]
    x0 = jnp.zeros((NP, EMBED), jnp.float32)
    x0 = x0.at[:USER_NUM].set(emb_user).at[USER_NUM:N_NODES].set(emb_item)
    dst = jnp.zeros((E_PAD,), jnp.int32).at[:n_edges].set(edge_index[0]).reshape(EDGE_ROWS, 128)
    src = jnp.zeros((E_PAD,), jnp.int32).at[:n_edges].set(edge_index[1]).reshape(EDGE_ROWS, 128)
    wbits = jax.lax.bitcast_convert_type(
        jnp.zeros((E_PAD,), jnp.float32).at[:n_edges].set(edge_weight),
        jnp.int32).reshape(EDGE_ROWS, 128)
    edata = jnp.stack([src, dst, wbits], axis=1)

    edp, cnt = _make_part()(edata)
    x1 = _make_layer(1.0 / 2)(x0, edp, cnt)
    x2 = _make_layer(1.0 / 3)(x1, edp, cnt)
    x3 = _make_layer(1.0 / 4)(x2, edp, cnt)

    u, i = _make_final()(x0, x1, x2, x3, userIdx, itemIdx)
    return _mlp(u, i, W1, b1, W2, b2, W3, b3)


# speedup vs baseline: 1.0846x; 1.0846x over previous
import functools

import jax
import jax.numpy as jnp
from jax import lax
from jax.experimental import pallas as pl
from jax.experimental.pallas import tpu as pltpu
from jax.experimental.pallas import tpu_sc as plsc

USER_NUM = 20000
N_NODES = 50000
EMBED = 64
NUM_LAYERS = 3
BATCH = 4096

NP = 50176
HALF = NP // 2
TRASH = 16
ACC_ROWS = HALF + TRASH
Z_SLICE = ACC_ROWS // 16
E_PAD = 802816
CHUNK = 128
EDGE_ROWS = E_PAD // 128
PROWS = EDGE_ROWS // 32
REGION_ROWS = PROWS + 4
NREGIONS = 64


_LANE_DNUMS = lax.GatherDimensionNumbers(
    offset_dims=(), collapsed_slice_dims=(0,), start_index_map=(0,))


def _lane_bcast(vec, lane):
    idx = jnp.full((16, 1), lane, jnp.int32)
    return lax.gather(vec, idx, _LANE_DNUMS, (1,),
                      mode=lax.GatherScatterMode.PROMISE_IN_BOUNDS)


def _part_body(edata_hbm, edp_hbm, cnt_hbm, ebuf, stg, cbuf, sem_i):
    c = lax.axis_index("c")
    s = lax.axis_index("s")
    w = s * 2 + c
    base = w * PROWS
    half_i = jnp.full((16,), HALF, jnp.int32)
    zero_v = jnp.zeros((16,), jnp.int32)

    def chunk_body(t, carry):
        pos0, pos1, row0, row1 = carry
        pltpu.async_copy(edata_hbm.at[base + t], ebuf, sem_i).wait()
        poss = [pos0, pos1]
        rows = [row0, row1]
        for h in range(2):
            pos = poss[h]
            for k in range(8):
                sv = ebuf[0, pl.ds(16 * k, 16)]
                dv = ebuf[1, pl.ds(16 * k, 16)] - (h * HALF)
                wv = ebuf[2, pl.ds(16 * k, 16)]
                ok = (dv >= 0) & (dv < half_i)
                plsc.store_compressed(stg.at[h, 0, pl.ds(pos, 16)], sv, mask=ok)
                plsc.store_compressed(stg.at[h, 1, pl.ds(pos, 16)], dv, mask=ok)
                plsc.store_compressed(stg.at[h, 2, pl.ds(pos, 16)], wv, mask=ok)
                pos = pos + jnp.max(plsc.all_reduce_population_count(ok))
            flush = pos >= 128
            rdst = (h * 32 + w) * REGION_ROWS + rows[h]

            @pl.when(flush)
            def _():
                pltpu.sync_copy(stg.at[h, :, pl.ds(0, 128)], edp_hbm.at[rdst])
                for a in range(3):
                    for i in range(16):
                        stg[h, a, pl.ds(16 * i, 16)] = stg[h, a, pl.ds(128 + 16 * i, 16)]

            poss[h] = jnp.where(flush, pos - 128, pos)
            rows[h] = jnp.where(flush, rows[h] + 1, rows[h])
        return (poss[0], poss[1], rows[0], rows[1])

    z32 = jnp.zeros((), jnp.int32)
    pos0, pos1, row0, row1 = lax.fori_loop(
        0, PROWS, chunk_body, (z32, z32, z32, z32))

    poss = [pos0, pos1]
    rowss = [row0, row1]
    for h in range(2):
        pos, row = poss[h], rowss[h]
        for a in range(3):
            for i in range(8):
                stg[h, a, pl.ds(pos + 16 * i, 16)] = zero_v
        rbase = (h * 32 + w) * REGION_ROWS
        pltpu.sync_copy(stg.at[h, :, pl.ds(0, 128)], edp_hbm.at[rbase + row])
        row = row + 1
        for a in range(3):
            for i in range(8):
                stg[h, a, pl.ds(16 * i, 16)] = zero_v
        for extra in range(3):
            pltpu.sync_copy(stg.at[h, :, pl.ds(0, 128)],
                            edp_hbm.at[rbase + row + extra])
        count = row + (row & 1)
        for j in range(1):
            cbuf[pl.ds(0, 16)] = jnp.full((16,), 0, jnp.int32) + count
        pltpu.sync_copy(cbuf, cnt_hbm.at[h * 32 + w])


@functools.lru_cache(maxsize=None)
def _make_part():
    mesh = plsc.VectorSubcoreMesh(core_axis_name="c", subcore_axis_name="s")
    return pl.kernel(
        _part_body,
        out_type=(jax.ShapeDtypeStruct((NREGIONS * REGION_ROWS, 3, 128), jnp.int32),
                  jax.ShapeDtypeStruct((NREGIONS, 16), jnp.int32)),
        mesh=mesh,
        scratch_types=[
            pltpu.VMEM((3, 128), jnp.int32),
            pltpu.VMEM((2, 3, 384), jnp.int32),
            pltpu.VMEM((16,), jnp.int32),
            pltpu.SemaphoreType.DMA,
        ],
        compiler_params=pltpu.CompilerParams(use_tc_tiling_on_sc=False, needs_layout_passes=False),
        name="lgcn_partition",
    )


def _layer_body(scale, x_hbm, edp_hbm, cnt_hbm, out_hbm,
                ev0, ev1, dlv, rows0, rows1, cbuf, acc, sem_i, sem_g0, sem_g1):
    c = lax.axis_index("c")
    s = lax.axis_index("s")

    def zz(e, _):
        z = jnp.zeros((16,), jnp.float32)
        for j in range(EMBED // 16):
            rows0[e, pl.ds(16 * j, 16)] = z
        return 0
    lax.fori_loop(0, CHUNK, zz, 0)
    for i in range(Z_SLICE // CHUNK):
        pltpu.sync_copy(rows0, acc.at[pl.ds(s * Z_SLICE + i * CHUNK, CHUNK)])
    rem = Z_SLICE % CHUNK
    if rem:
        pltpu.sync_copy(rows0.at[pl.ds(0, rem)],
                        acc.at[pl.ds(s * Z_SLICE + (Z_SLICE // CHUNK) * CHUNK, rem)])
    plsc.subcore_barrier()

    bufs = ((ev0, rows0, sem_g0), (ev1, rows1, sem_g1))

    for ri in range(2):
        reg = s * 2 + ri
        rbase = (c * 32 + reg) * REGION_ROWS
        pltpu.sync_copy(cnt_hbm.at[c * 32 + reg], cbuf)
        nchunks = jnp.max(cbuf[pl.ds(0, 16)])

        pltpu.async_copy(edp_hbm.at[rbase], ev0, sem_i).wait()
        pltpu.async_copy(x_hbm.at[ev0.at[0]], rows0, sem_g0)
        pltpu.async_copy(edp_hbm.at[rbase + 1], ev1, sem_i)

        def step(t, cur, nxt):
            ebuf, rows, sem_g = cur
            ebuf_n, rows_n, sem_g_n = nxt
            pltpu.make_async_copy(edp_hbm.at[rbase + t + 1], ebuf_n, sem_i).wait()
            pltpu.async_copy(x_hbm.at[ebuf_n.at[0]], rows_n, sem_g_n)
            for k in range(8):
                dlv[0, pl.ds(16 * k, 16)] = ebuf[1, pl.ds(16 * k, 16)]
            wvecs = [plsc.bitcast(ebuf[2, pl.ds(16 * b, 16)], jnp.float32) * scale
                     for b in range(8)]
            pltpu.make_async_copy(x_hbm.at[ebuf.at[0]], rows, sem_g).wait()
            pltpu.async_copy(edp_hbm.at[rbase + t + 2], ebuf, sem_i)
            for b in range(8):
                for l in range(16):
                    wb = _lane_bcast(wvecs[b], l)
                    e = 16 * b + l
                    for j in range(EMBED // 16):
                        rows[e, pl.ds(16 * j, 16)] = rows[e, pl.ds(16 * j, 16)] * wb
            pltpu.sync_copy(rows, acc.at[dlv.at[0]], add=True)

        def pair_body(i, _):
            t = i * 2
            step(t, bufs[0], bufs[1])
            step(t + 1, bufs[1], bufs[0])
            return 0
        lax.fori_loop(0, 1, pair_body, 0)
        lax.fori_loop(1, lax.max(nchunks, 2) // 2, pair_body, 0)

        pltpu.make_async_copy(x_hbm.at[ev0.at[0]], rows0, sem_g0).wait()
        pltpu.make_async_copy(edp_hbm.at[rbase], ev1, sem_i).wait()

    plsc.subcore_barrier()

    wb_rows = HALF // 16
    pltpu.sync_copy(acc.at[pl.ds(s * wb_rows, wb_rows)],
                    out_hbm.at[pl.ds(c * HALF + s * wb_rows, wb_rows)])


@functools.lru_cache(maxsize=None)
def _make_layer(scale):
    mesh = plsc.VectorSubcoreMesh(core_axis_name="c", subcore_axis_name="s")
    return pl.kernel(
        functools.partial(_layer_body, scale),
        out_type=jax.ShapeDtypeStruct((NP, EMBED), jnp.float32),
        mesh=mesh,
        scratch_types=[
            pltpu.VMEM((3, 128), jnp.int32),
            pltpu.VMEM((3, 128), jnp.int32),
            pltpu.VMEM((1, 128), jnp.int32),
            pltpu.VMEM((CHUNK, EMBED), jnp.float32),
            pltpu.VMEM((CHUNK, EMBED), jnp.float32),
            pltpu.VMEM((16,), jnp.int32),
            pltpu.VMEM_SHARED((ACC_ROWS, EMBED), jnp.float32),
            pltpu.SemaphoreType.DMA,
            pltpu.SemaphoreType.DMA,
            pltpu.SemaphoreType.DMA,
        ],
        compiler_params=pltpu.CompilerParams(use_tc_tiling_on_sc=False, needs_layout_passes=False),
        name=f"lgcn_layer_{int(1.0/scale)}",
    )


def _final_body(x0, x1, x2, x3, uidx_hbm, iidx_hbm, u_hbm, i_hbm,
                idxv, g0, g1, g2, g3, sem):
    c = lax.axis_index("c")
    s = lax.axis_index("s")
    wid = s * 2 + c
    base = wid * (BATCH // 32)

    def do(idx_hbm, off, out_hbm):
        pltpu.sync_copy(idx_hbm.at[pl.ds(base, BATCH // 32)], idxv)
        if off:
            offv = jnp.full((16,), off, jnp.int32)
            for k in range(BATCH // 32 // 16):
                idxv[pl.ds(16 * k, 16)] = idxv[pl.ds(16 * k, 16)] + offv
        cps = [pltpu.async_copy(x.at[idxv], g, sem)
               for x, g in ((x0, g0), (x1, g1), (x2, g2), (x3, g3))]
        for cp in cps:
            cp.wait()

        def sum_body(e, _):
            for j in range(EMBED // 16):
                d = pl.ds(16 * j, 16)
                g0[e, d] = g0[e, d] + g1[e, d] + g2[e, d] + g3[e, d]
            return 0
        lax.fori_loop(0, BATCH // 32, sum_body, 0)
        pltpu.sync_copy(g0, out_hbm.at[pl.ds(base, BATCH // 32)])

    do(uidx_hbm, 0, u_hbm)
    do(iidx_hbm, USER_NUM, i_hbm)


@functools.lru_cache(maxsize=None)
def _make_final():
    mesh = plsc.VectorSubcoreMesh(core_axis_name="c", subcore_axis_name="s")
    return pl.kernel(
        _final_body,
        out_type=(jax.ShapeDtypeStruct((BATCH, EMBED), jnp.float32),
                  jax.ShapeDtypeStruct((BATCH, EMBED), jnp.float32)),
        mesh=mesh,
        scratch_types=[
            pltpu.VMEM((BATCH // 32,), jnp.int32),
            pltpu.VMEM((BATCH // 32, EMBED), jnp.float32),
            pltpu.VMEM((BATCH // 32, EMBED), jnp.float32),
            pltpu.VMEM((BATCH // 32, EMBED), jnp.float32),
            pltpu.VMEM((BATCH // 32, EMBED), jnp.float32),
            pltpu.SemaphoreType.DMA,
        ],
        compiler_params=pltpu.CompilerParams(use_tc_tiling_on_sc=False, needs_layout_passes=False),
        name="lgcn_final_gather",
    )


def _mlp_body(u_ref, i_ref, w1u_ref, w1i_ref, b1_ref, w2_ref, b2_ref, w3_ref, b3_ref, o_ref):
    h = jnp.dot(u_ref[...], w1u_ref[...], preferred_element_type=jnp.float32)
    h += jnp.dot(i_ref[...], w1i_ref[...], preferred_element_type=jnp.float32)
    h = jax.nn.relu(h + b1_ref[...])
    h2 = jnp.dot(h, w2_ref[...], preferred_element_type=jnp.float32) + b2_ref[...]
    o_ref[...] = jnp.dot(h2, w3_ref[...], preferred_element_type=jnp.float32) + b3_ref[...]


def _mlp(u, i, W1, b1, W2, b2, W3, b3):
    out = pl.pallas_call(
        _mlp_body,
        out_shape=jax.ShapeDtypeStruct((BATCH, 1), jnp.float32),
    )(u, i, W1[:EMBED], W1[EMBED:], b1[None, :], W2, b2[None, :], W3, b3[None, :])
    return out.reshape(-1)


def kernel(userIdx, itemIdx, edge_index, edge_weight, emb_user, emb_item, W1, b1, W2, b2, W3, b3):
    n_edges = edge_weight.shape[0]
    x0 = jnp.zeros((NP, EMBED), jnp.float32)
    x0 = x0.at[:USER_NUM].set(emb_user).at[USER_NUM:N_NODES].set(emb_item)
    dst = jnp.zeros((E_PAD,), jnp.int32).at[:n_edges].set(edge_index[0]).reshape(EDGE_ROWS, 128)
    src = jnp.zeros((E_PAD,), jnp.int32).at[:n_edges].set(edge_index[1]).reshape(EDGE_ROWS, 128)
    wbits = jax.lax.bitcast_convert_type(
        jnp.zeros((E_PAD,), jnp.float32).at[:n_edges].set(edge_weight),
        jnp.int32).reshape(EDGE_ROWS, 128)
    edata = jnp.stack([src, dst, wbits], axis=1)

    edp, cnt = _make_part()(edata)
    x1 = _make_layer(1.0 / 2)(x0, edp, cnt)
    x2 = _make_layer(1.0 / 3)(x1, edp, cnt)
    x3 = _make_layer(1.0 / 4)(x2, edp, cnt)

    u, i = _make_final()(x0, x1, x2, x3, userIdx, itemIdx)
    return _mlp(u, i, W1, b1, W2, b2, W3, b3)


# speedup vs baseline: 1.1285x; 1.0405x over previous
import functools

import jax
import jax.numpy as jnp
from jax import lax
from jax.experimental import pallas as pl
from jax.experimental.pallas import tpu as pltpu
from jax.experimental.pallas import tpu_sc as plsc

USER_NUM = 20000
N_NODES = 50000
EMBED = 64
NUM_LAYERS = 3
BATCH = 4096

NP = 50176
HALF = NP // 2
TRASH = 16
ACC_ROWS = HALF + TRASH
Z_SLICE = ACC_ROWS // 16
E_PAD = 802816
CHUNK = 128
EDGE_ROWS = E_PAD // 128
PROWS = EDGE_ROWS // 32
REGION_ROWS = PROWS + 4
NREGIONS = 64


_LANE_DNUMS = lax.GatherDimensionNumbers(
    offset_dims=(), collapsed_slice_dims=(0,), start_index_map=(0,))


def _lane_bcast(vec, lane):
    idx = jnp.full((16, 1), lane, jnp.int32)
    return lax.gather(vec, idx, _LANE_DNUMS, (1,),
                      mode=lax.GatherScatterMode.PROMISE_IN_BOUNDS)


def _part_body(edata_hbm, edp_hbm, cnt_hbm, ebuf, ebuf2, stg, cbuf, sem_i, sem_i2):
    c = lax.axis_index("c")
    s = lax.axis_index("s")
    w = s * 2 + c
    base = w * PROWS
    half_i = jnp.full((16,), HALF, jnp.int32)
    zero_v = jnp.zeros((16,), jnp.int32)

    def do_chunk(t, carry, ebuf, sem):
        pos0, pos1, row0, row1 = carry
        pltpu.make_async_copy(edata_hbm.at[base + t], ebuf, sem).wait()

        poss = [pos0, pos1]
        rows = [row0, row1]
        for h in range(2):
            pos = poss[h]
            for k in range(8):
                sv = ebuf[0, pl.ds(16 * k, 16)]
                dv = ebuf[1, pl.ds(16 * k, 16)] - (h * HALF)
                wv = ebuf[2, pl.ds(16 * k, 16)]
                ok = (dv >= 0) & (dv < half_i)
                plsc.store_compressed(stg.at[h, 0, pl.ds(pos, 16)], sv, mask=ok)
                plsc.store_compressed(stg.at[h, 1, pl.ds(pos, 16)], dv, mask=ok)
                plsc.store_compressed(stg.at[h, 2, pl.ds(pos, 16)], wv, mask=ok)
                pos = pos + jnp.max(plsc.all_reduce_population_count(ok))
            flush = pos >= 128
            rdst = (h * 32 + w) * REGION_ROWS + rows[h]

            @pl.when(flush)
            def _():
                pltpu.sync_copy(stg.at[h, :, pl.ds(0, 128)], edp_hbm.at[rdst])
                for a in range(3):
                    for i in range(16):
                        stg[h, a, pl.ds(16 * i, 16)] = stg[h, a, pl.ds(128 + 16 * i, 16)]

            poss[h] = jnp.where(flush, pos - 128, pos)
            rows[h] = jnp.where(flush, rows[h] + 1, rows[h])
        return (poss[0], poss[1], rows[0], rows[1])

    def pair_body(i, carry):
        t = i * 2
        carry = do_chunk(t, carry, ebuf, sem_i)
        pltpu.async_copy(edata_hbm.at[base + jnp.minimum(t + 2, PROWS - 1)],
                         ebuf, sem_i)
        carry = do_chunk(t + 1, carry, ebuf2, sem_i2)
        pltpu.async_copy(edata_hbm.at[base + jnp.minimum(t + 3, PROWS - 1)],
                         ebuf2, sem_i2)
        return carry

    z32 = jnp.zeros((), jnp.int32)
    pltpu.async_copy(edata_hbm.at[base], ebuf, sem_i)
    pltpu.async_copy(edata_hbm.at[base + 1], ebuf2, sem_i2)
    pos0, pos1, row0, row1 = lax.fori_loop(
        0, PROWS // 2, pair_body, (z32, z32, z32, z32))
    pltpu.make_async_copy(edata_hbm.at[base], ebuf2, sem_i2).wait()
    pltpu.make_async_copy(edata_hbm.at[base], ebuf, sem_i).wait()

    poss = [pos0, pos1]
    rowss = [row0, row1]
    for h in range(2):
        pos, row = poss[h], rowss[h]
        for a in range(3):
            for i in range(8):
                stg[h, a, pl.ds(pos + 16 * i, 16)] = zero_v
        rbase = (h * 32 + w) * REGION_ROWS
        pltpu.sync_copy(stg.at[h, :, pl.ds(0, 128)], edp_hbm.at[rbase + row])
        row = row + 1
        for a in range(3):
            for i in range(8):
                stg[h, a, pl.ds(16 * i, 16)] = zero_v
        for extra in range(3):
            pltpu.sync_copy(stg.at[h, :, pl.ds(0, 128)],
                            edp_hbm.at[rbase + row + extra])
        count = row + (row & 1)
        for j in range(1):
            cbuf[pl.ds(0, 16)] = jnp.full((16,), 0, jnp.int32) + count
        pltpu.sync_copy(cbuf, cnt_hbm.at[h * 32 + w])


@functools.lru_cache(maxsize=None)
def _make_part():
    mesh = plsc.VectorSubcoreMesh(core_axis_name="c", subcore_axis_name="s")
    return pl.kernel(
        _part_body,
        out_type=(jax.ShapeDtypeStruct((NREGIONS * REGION_ROWS, 3, 128), jnp.int32),
                  jax.ShapeDtypeStruct((NREGIONS, 16), jnp.int32)),
        mesh=mesh,
        scratch_types=[
            pltpu.VMEM((3, 128), jnp.int32),
            pltpu.VMEM((3, 128), jnp.int32),
            pltpu.VMEM((2, 3, 384), jnp.int32),
            pltpu.VMEM((16,), jnp.int32),
            pltpu.SemaphoreType.DMA,
            pltpu.SemaphoreType.DMA,
        ],
        compiler_params=pltpu.CompilerParams(use_tc_tiling_on_sc=False, needs_layout_passes=False),
        name="lgcn_partition",
    )


def _layer_body(scale, x_hbm, edp_hbm, cnt_hbm, out_hbm,
                ev0, ev1, dlv, rows0, rows1, cbuf, acc, sem_i, sem_g0, sem_g1):
    c = lax.axis_index("c")
    s = lax.axis_index("s")

    def zz(e, _):
        z = jnp.zeros((16,), jnp.float32)
        for j in range(EMBED // 16):
            rows0[e, pl.ds(16 * j, 16)] = z
        return 0
    lax.fori_loop(0, CHUNK, zz, 0)
    for i in range(Z_SLICE // CHUNK):
        pltpu.sync_copy(rows0, acc.at[pl.ds(s * Z_SLICE + i * CHUNK, CHUNK)])
    rem = Z_SLICE % CHUNK
    if rem:
        pltpu.sync_copy(rows0.at[pl.ds(0, rem)],
                        acc.at[pl.ds(s * Z_SLICE + (Z_SLICE // CHUNK) * CHUNK, rem)])
    plsc.subcore_barrier()

    bufs = ((ev0, rows0, sem_g0), (ev1, rows1, sem_g1))

    for ri in range(2):
        reg = s * 2 + ri
        rbase = (c * 32 + reg) * REGION_ROWS
        pltpu.sync_copy(cnt_hbm.at[c * 32 + reg], cbuf)
        nchunks = jnp.max(cbuf[pl.ds(0, 16)])

        pltpu.async_copy(edp_hbm.at[rbase], ev0, sem_i).wait()
        pltpu.async_copy(x_hbm.at[ev0.at[0]], rows0, sem_g0)
        pltpu.async_copy(edp_hbm.at[rbase + 1], ev1, sem_i)

        def step(t, cur, nxt):
            ebuf, rows, sem_g = cur
            ebuf_n, rows_n, sem_g_n = nxt
            pltpu.make_async_copy(edp_hbm.at[rbase + t + 1], ebuf_n, sem_i).wait()
            pltpu.async_copy(x_hbm.at[ebuf_n.at[0]], rows_n, sem_g_n)
            for k in range(8):
                dlv[0, pl.ds(16 * k, 16)] = ebuf[1, pl.ds(16 * k, 16)]
            wvecs = [plsc.bitcast(ebuf[2, pl.ds(16 * b, 16)], jnp.float32) * scale
                     for b in range(8)]
            pltpu.make_async_copy(x_hbm.at[ebuf.at[0]], rows, sem_g).wait()
            pltpu.async_copy(edp_hbm.at[rbase + t + 2], ebuf, sem_i)
            for b in range(8):
                for l in range(16):
                    wb = _lane_bcast(wvecs[b], l)
                    e = 16 * b + l
                    for j in range(EMBED // 16):
                        rows[e, pl.ds(16 * j, 16)] = rows[e, pl.ds(16 * j, 16)] * wb
            pltpu.sync_copy(rows, acc.at[dlv.at[0]], add=True)

        def pair_body(i, _):
            t = i * 2
            step(t, bufs[0], bufs[1])
            step(t + 1, bufs[1], bufs[0])
            return 0
        lax.fori_loop(0, 1, pair_body, 0)
        lax.fori_loop(1, lax.max(nchunks, 2) // 2, pair_body, 0)

        pltpu.make_async_copy(x_hbm.at[ev0.at[0]], rows0, sem_g0).wait()
        pltpu.make_async_copy(edp_hbm.at[rbase], ev1, sem_i).wait()

    plsc.subcore_barrier()

    wb_rows = HALF // 16
    pltpu.sync_copy(acc.at[pl.ds(s * wb_rows, wb_rows)],
                    out_hbm.at[pl.ds(c * HALF + s * wb_rows, wb_rows)])


@functools.lru_cache(maxsize=None)
def _make_layer(scale):
    mesh = plsc.VectorSubcoreMesh(core_axis_name="c", subcore_axis_name="s")
    return pl.kernel(
        functools.partial(_layer_body, scale),
        out_type=jax.ShapeDtypeStruct((NP, EMBED), jnp.float32),
        mesh=mesh,
        scratch_types=[
            pltpu.VMEM((3, 128), jnp.int32),
            pltpu.VMEM((3, 128), jnp.int32),
            pltpu.VMEM((1, 128), jnp.int32),
            pltpu.VMEM((CHUNK, EMBED), jnp.float32),
            pltpu.VMEM((CHUNK, EMBED), jnp.float32),
            pltpu.VMEM((16,), jnp.int32),
            pltpu.VMEM_SHARED((ACC_ROWS, EMBED), jnp.float32),
            pltpu.SemaphoreType.DMA,
            pltpu.SemaphoreType.DMA,
            pltpu.SemaphoreType.DMA,
        ],
        compiler_params=pltpu.CompilerParams(use_tc_tiling_on_sc=False, needs_layout_passes=False),
        name=f"lgcn_layer_{int(1.0/scale)}",
    )


def _final_body(x0, x1, x2, x3, uidx_hbm, iidx_hbm, u_hbm, i_hbm,
                idxv, g0, g1, g2, g3, sem):
    c = lax.axis_index("c")
    s = lax.axis_index("s")
    wid = s * 2 + c
    base = wid * (BATCH // 32)

    def do(idx_hbm, off, out_hbm):
        pltpu.sync_copy(idx_hbm.at[pl.ds(base, BATCH // 32)], idxv)
        if off:
            offv = jnp.full((16,), off, jnp.int32)
            for k in range(BATCH // 32 // 16):
                idxv[pl.ds(16 * k, 16)] = idxv[pl.ds(16 * k, 16)] + offv
        cps = [pltpu.async_copy(x.at[idxv], g, sem)
               for x, g in ((x0, g0), (x1, g1), (x2, g2), (x3, g3))]
        for cp in cps:
            cp.wait()

        def sum_body(e, _):
            for j in range(EMBED // 16):
                d = pl.ds(16 * j, 16)
                g0[e, d] = g0[e, d] + g1[e, d] + g2[e, d] + g3[e, d]
            return 0
        lax.fori_loop(0, BATCH // 32, sum_body, 0)
        pltpu.sync_copy(g0, out_hbm.at[pl.ds(base, BATCH // 32)])

    do(uidx_hbm, 0, u_hbm)
    do(iidx_hbm, USER_NUM, i_hbm)


@functools.lru_cache(maxsize=None)
def _make_final():
    mesh = plsc.VectorSubcoreMesh(core_axis_name="c", subcore_axis_name="s")
    return pl.kernel(
        _final_body,
        out_type=(jax.ShapeDtypeStruct((BATCH, EMBED), jnp.float32),
                  jax.ShapeDtypeStruct((BATCH, EMBED), jnp.float32)),
        mesh=mesh,
        scratch_types=[
            pltpu.VMEM((BATCH // 32,), jnp.int32),
            pltpu.VMEM((BATCH // 32, EMBED), jnp.float32),
            pltpu.VMEM((BATCH // 32, EMBED), jnp.float32),
            pltpu.VMEM((BATCH // 32, EMBED), jnp.float32),
            pltpu.VMEM((BATCH // 32, EMBED), jnp.float32),
            pltpu.SemaphoreType.DMA,
        ],
        compiler_params=pltpu.CompilerParams(use_tc_tiling_on_sc=False, needs_layout_passes=False),
        name="lgcn_final_gather",
    )


def _mlp_body(u_ref, i_ref, w1u_ref, w1i_ref, b1_ref, w2_ref, b2_ref, w3_ref, b3_ref, o_ref):
    h = jnp.dot(u_ref[...], w1u_ref[...], preferred_element_type=jnp.float32)
    h += jnp.dot(i_ref[...], w1i_ref[...], preferred_element_type=jnp.float32)
    h = jax.nn.relu(h + b1_ref[...])
    h2 = jnp.dot(h, w2_ref[...], preferred_element_type=jnp.float32) + b2_ref[...]
    o_ref[...] = jnp.dot(h2, w3_ref[...], preferred_element_type=jnp.float32) + b3_ref[...]


def _mlp(u, i, W1, b1, W2, b2, W3, b3):
    out = pl.pallas_call(
        _mlp_body,
        out_shape=jax.ShapeDtypeStruct((BATCH, 1), jnp.float32),
    )(u, i, W1[:EMBED], W1[EMBED:], b1[None, :], W2, b2[None, :], W3, b3[None, :])
    return out.reshape(-1)


def kernel(userIdx, itemIdx, edge_index, edge_weight, emb_user, emb_item, W1, b1, W2, b2, W3, b3):
    n_edges = edge_weight.shape[0]
    x0 = jnp.zeros((NP, EMBED), jnp.float32)
    x0 = x0.at[:USER_NUM].set(emb_user).at[USER_NUM:N_NODES].set(emb_item)
    dst = jnp.zeros((E_PAD,), jnp.int32).at[:n_edges].set(edge_index[0]).reshape(EDGE_ROWS, 128)
    src = jnp.zeros((E_PAD,), jnp.int32).at[:n_edges].set(edge_index[1]).reshape(EDGE_ROWS, 128)
    wbits = jax.lax.bitcast_convert_type(
        jnp.zeros((E_PAD,), jnp.float32).at[:n_edges].set(edge_weight),
        jnp.int32).reshape(EDGE_ROWS, 128)
    edata = jnp.stack([src, dst, wbits], axis=1)

    edp, cnt = _make_part()(edata)
    x1 = _make_layer(1.0 / 2)(x0, edp, cnt)
    x2 = _make_layer(1.0 / 3)(x1, edp, cnt)
    x3 = _make_layer(1.0 / 4)(x2, edp, cnt)

    u, i = _make_final()(x0, x1, x2, x3, userIdx, itemIdx)
    return _mlp(u, i, W1, b1, W2, b2, W3, b3)
